# baseline (speedup 1.0000x reference)
import functools

import jax
import jax.numpy as jnp
from jax import lax
from jax.experimental import pallas as pl
from jax.experimental.pallas import tpu as pltpu
from jax.experimental.pallas import tpu_sc as plsc

V = 1000000
D = 64
R = 8
SCALING = 2.0

NC, NS = 2, 16
NW = NC * NS
BB, LL = 1024, 200
TOK = BB * LL
PW = TOK // NW
XR = BB // NW
CHUNKS = ((0, 104), (104, 96))
TG = LL
NG = PW // TG


def _sc_body(x_ref, w_ref, a_ref, b_ref, out_ref,
             idx_v, wrows_v, arows_v, bt_v, wsem, asem, osem):
    cid = lax.axis_index("c")
    sid = lax.axis_index("s")
    wid = sid * NC + cid
    tok0 = wid * PW

    pltpu.sync_copy(x_ref.at[pl.ds(wid * XR, XR)], idx_v)
    pltpu.sync_copy(b_ref, bt_v)

    bts = [[bt_v[r, pl.ds(k * 16, 16)] for k in range(D // 16)]
           for r in range(R)]
    rconsts = [jnp.full((16,), r, jnp.int32) for r in range(R)]

    def gathers(g, slot):
        cps = []
        for off, sz in CHUNKS:
            isl = idx_v.at[g, pl.ds(off, sz)]
            cps.append(pltpu.make_async_copy(
                w_ref.at[isl], wrows_v.at[slot, pl.ds(off, sz)], wsem))
            for r in range(R):
                cps.append(pltpu.make_async_copy(
                    a_ref.at[r].at[isl],
                    arows_v.at[slot, r, pl.ds(off, sz)], asem))
        return cps

    def outcopy(g, slot):
        return pltpu.make_async_copy(
            wrows_v.at[slot], out_ref.at[pl.ds(tok0 + g * TG, TG)], osem)

    def compute(slot):
        def tok_body(t, tc):
            accs = [wrows_v[slot, t, pl.ds(k * 16, 16)]
                    for k in range(D // 16)]
            tvec = jnp.full((16,), t, jnp.int32)
            for r in range(R):
                ar = plsc.load_gather(arows_v.at[slot], [rconsts[r], tvec])
                for k in range(D // 16):
                    accs[k] = accs[k] + ar * bts[r][k]
            for k in range(D // 16):
                wrows_v[slot, t, pl.ds(k * 16, 16)] = accs[k]
            return tc
        lax.fori_loop(0, TG, tok_body, 0, unroll=2)

    for cp in gathers(0, 0):
        cp.start()

    def group_body(g, carry):
        slot = jnp.bitwise_and(g, 1)

        @pl.when(g < NG - 1)
        def _fire_next():
            @pl.when(g >= 1)
            def _drain_prev_write():
                outcopy(g - 1, 1 - slot).wait()
            for cp in gathers(g + 1, 1 - slot):
                cp.start()

        for cp in gathers(g, slot):
            cp.wait()
        compute(slot)
        outcopy(g, slot).start()
        return carry

    lax.fori_loop(0, NG, group_body, 0)
    outcopy(NG - 2, 0).wait()
    outcopy(NG - 1, 1).wait()


@functools.cache
def _sc_lora_embed():
    return functools.partial(
        pl.kernel,
        out_type=jax.ShapeDtypeStruct((TOK, D), jnp.float32),
        mesh=plsc.VectorSubcoreMesh(core_axis_name="c", subcore_axis_name="s"),
        compiler_params=pltpu.CompilerParams(
            use_tc_tiling_on_sc=False, needs_layout_passes=False),
        scratch_types=[
            pltpu.VMEM((XR, LL), jnp.int32),
            pltpu.VMEM((2, TG, D), jnp.float32),
            pltpu.VMEM((2, R, TG), jnp.float32),
            pltpu.VMEM((R, D), jnp.float32),
            pltpu.SemaphoreType.DMA,
            pltpu.SemaphoreType.DMA,
            pltpu.SemaphoreType.DMA,
        ],
    )(_sc_body)


@jax.jit
def kernel(x, weight, lora_A, lora_B):
    bt = (SCALING * lora_B).T
    out = _sc_lora_embed()(x, weight, lora_A, bt)
    return out.reshape(BB, LL, D)


# speedup vs baseline: 1.1061x; 1.1061x over previous
import functools

import jax
import jax.numpy as jnp
from jax import lax
from jax.experimental import pallas as pl
from jax.experimental.pallas import tpu as pltpu
from jax.experimental.pallas import tpu_sc as plsc

V = 1000000
D = 64
R = 8
SCALING = 2.0

NC, NS = 2, 16
NW = NC * NS
BB, LL = 1024, 200
TOK = BB * LL
PW = TOK // NW
CH = 128
TG = 2 * CH
NG = PW // TG
NSLOT = 4


def _sc_body(x_ref, w_ref, a_ref, b_ref, out_ref,
             idx_v, wrows_v, arows_v, bt_v, wsem, asem, osem):
    cid = lax.axis_index("c")
    sid = lax.axis_index("s")
    wid = sid * NC + cid
    tok0 = wid * PW

    pltpu.sync_copy(x_ref.at[pl.ds(tok0, PW)], idx_v)
    pltpu.sync_copy(b_ref, bt_v)

    bts = [[bt_v[r, pl.ds(k * 16, 16)] for k in range(D // 16)]
           for r in range(R)]
    rconsts = [jnp.full((16,), r, jnp.int32) for r in range(R)]

    def gathers(g, slot):
        cps = []
        for c in range(2):
            isl = idx_v.at[pl.ds((g * 2 + c) * CH, CH)]
            cps.append(pltpu.make_async_copy(
                w_ref.at[isl], wrows_v.at[slot, pl.ds(c * CH, CH)], wsem))
            cps.append(pltpu.make_async_copy(
                a_ref.at[isl], arows_v.at[slot, pl.ds(c * CH, CH)], asem))
        return cps

    def outcopy(g, slot):
        return pltpu.make_async_copy(
            wrows_v.at[slot], out_ref.at[pl.ds(tok0 + g * TG, TG)], osem)

    def compute(slot):
        def tok_body(t, tc):
            accs = [wrows_v[slot, t, pl.ds(k * 16, 16)]
                    for k in range(D // 16)]
            tvec = jnp.full((16,), t, jnp.int32)
            for r in range(R):
                ar = plsc.load_gather(arows_v.at[slot], [tvec, rconsts[r]])
                for k in range(D // 16):
                    accs[k] = accs[k] + ar * bts[r][k]
            for k in range(D // 16):
                wrows_v[slot, t, pl.ds(k * 16, 16)] = accs[k]
            return tc
        lax.fori_loop(0, TG, tok_body, 0, unroll=2)

    for gp in range(NSLOT - 1):
        for cp in gathers(gp, gp):
            cp.start()

    def group_body(g, carry):
        slot = lax.rem(g, NSLOT)

        @pl.when(g + NSLOT - 1 < NG)
        def _fire_ahead():
            @pl.when(g >= 1)
            def _drain_prev_write():
                outcopy(g - 1, lax.rem(g - 1, NSLOT)).wait()
            for cp in gathers(g + NSLOT - 1, lax.rem(g + NSLOT - 1, NSLOT)):
                cp.start()

        for cp in gathers(g, slot):
            cp.wait()
        compute(slot)
        outcopy(g, slot).start()
        return carry

    lax.fori_loop(0, NG, group_body, 0)
    for g in range(NG - NSLOT, NG):
        outcopy(g, g % NSLOT).wait()


@functools.cache
def _sc_lora_embed():
    return functools.partial(
        pl.kernel,
        out_type=jax.ShapeDtypeStruct((TOK, D), jnp.float32),
        mesh=plsc.VectorSubcoreMesh(core_axis_name="c", subcore_axis_name="s"),
        compiler_params=pltpu.CompilerParams(
            use_tc_tiling_on_sc=False, needs_layout_passes=False),
        scratch_types=[
            pltpu.VMEM((PW,), jnp.int32),
            pltpu.VMEM((NSLOT, TG, D), jnp.float32),
            pltpu.VMEM((NSLOT, TG, R), jnp.float32),
            pltpu.VMEM((R, D), jnp.float32),
            pltpu.SemaphoreType.DMA,
            pltpu.SemaphoreType.DMA,
            pltpu.SemaphoreType.DMA,
        ],
    )(_sc_body)


@jax.jit
def kernel(x, weight, lora_A, lora_B):
    x1 = x.reshape(TOK)
    a_t = (SCALING * lora_A).T
    bt = lora_B.T
    out = _sc_lora_embed()(x1, weight, a_t, bt)
    return out.reshape(BB, LL, D)
